# baseline (speedup 1.0000x reference)
import math

import jax
import jax.numpy as jnp
from jax.experimental import pallas as pl
from jax.experimental.pallas import tpu as pltpu

_C0 = math.sqrt(2.0 / math.pi)
_C1 = 0.044715


def _gelu_block(x_ref, o_ref):
    x = x_ref[...]
    inner = _C0 * (x + _C1 * (x * x * x))
    o_ref[...] = 0.5 * x * (1.0 + jnp.tanh(inner))


def kernel(x, log_tau, log_blend):
    b, t, d = x.shape
    rows = b * t
    x2 = x.reshape(rows, d)
    block_rows = 512
    out = pl.pallas_call(
        _gelu_block,
        grid=(rows // block_rows,),
        in_specs=[pl.BlockSpec((block_rows, d), lambda i: (i, 0))],
        out_specs=pl.BlockSpec((block_rows, d), lambda i: (i, 0)),
        out_shape=jax.ShapeDtypeStruct((rows, d), x.dtype),
        compiler_params=pltpu.CompilerParams(
            dimension_semantics=("arbitrary",),
        ),
    )(x2)
    return out.reshape(b, t, d)
